# Initial kernel scaffold; baseline (speedup 1.0000x reference)
import jax
import jax.numpy as jnp
from jax.experimental import pallas as pl


def kernel(Q, K, V):
    raise NotImplementedError("write your pallas kernel here")


# speedup vs baseline: 4.2077x; 4.2077x over previous
import functools
import math

import jax
import jax.numpy as jnp
from jax.experimental import pallas as pl
from jax.experimental.pallas import tpu as pltpu

_FACTOR = 5.0
_EPS = 1e-09


def _prob_sparse_kernel(q_ref, k_ref, v_ref, out_ref, *, k_top, k_pad, L, D):
    q = q_ref[0]
    kk = k_ref[0]
    v = v_ref[0]
    scale = 1.0 / math.sqrt(D)

    s = jnp.dot(q, kk.T, preferred_element_type=jnp.float32) * scale
    m_max = jnp.max(s, axis=-1, keepdims=True)
    m_mean = jnp.sum(s, axis=-1, keepdims=True) * (1.0 / L)
    row = jnp.reshape(m_max - m_mean, (1, L))

    lane_iota = jax.lax.broadcasted_iota(jnp.int32, (1, L), 1)
    neg_inf = jnp.float32(-jnp.inf)

    idx_rows = []
    work = row
    for _ in range(k_top):
        i_j = jnp.argmax(work, axis=-1).reshape(1, 1).astype(jnp.int32)
        idx_rows.append(i_j)
        work = jnp.where(lane_iota == i_j, neg_inf, work)
    for _ in range(k_pad - k_top):
        idx_rows.append(jnp.full((1, 1), -1, dtype=jnp.int32))
    idx_col = jnp.concatenate(idx_rows, axis=0)

    onehot = (jax.lax.broadcasted_iota(jnp.int32, (k_pad, L), 1)
              == idx_col).astype(jnp.float32)

    qs = jnp.dot(onehot, q, preferred_element_type=jnp.float32)
    ssp = jnp.dot(qs, kk.T, preferred_element_type=jnp.float32) * scale
    smax = jnp.max(ssp, axis=-1, keepdims=True)
    e = jnp.exp(ssp - smax)
    att = e / jnp.sum(e, axis=-1, keepdims=True)
    ctx = jnp.dot(att, v, preferred_element_type=jnp.float32)

    v_mean = jnp.mean(v, axis=0, keepdims=True)
    delta = ctx - v_mean
    scat = jax.lax.dot_general(
        onehot, delta, (((0,), (0,)), ((), ())),
        preferred_element_type=jnp.float32)
    out_ref[0] = scat + v_mean


def kernel(Q, K, V):
    B, L, D = Q.shape
    k_top = min(L, max(1, int(_FACTOR * math.log(L + _EPS))))
    k_pad = max(8, ((k_top + 7) // 8) * 8)

    spec = pl.BlockSpec((1, L, D), lambda b: (b, 0, 0))
    return pl.pallas_call(
        functools.partial(_prob_sparse_kernel, k_top=k_top, k_pad=k_pad,
                          L=L, D=D),
        grid=(B,),
        in_specs=[spec, spec, spec],
        out_specs=spec,
        out_shape=jax.ShapeDtypeStruct((B, L, D), jnp.float32),
    )(Q, K, V)


# speedup vs baseline: 4.2386x; 1.0074x over previous
import functools
import math

import jax
import jax.numpy as jnp
from jax.experimental import pallas as pl
from jax.experimental.pallas import tpu as pltpu

_FACTOR = 5.0
_EPS = 1e-09


def _prob_sparse_kernel(q_ref, k_ref, v_ref, out_ref, *, k_top, k_pad, L, D,
                        n_chunks):
    q = q_ref[0]
    kk = k_ref[0]
    v = v_ref[0]
    scale = 1.0 / math.sqrt(D)

    qT = q.T

    ksum = jnp.sum(kk, axis=0, keepdims=True)
    rowsum = jnp.dot(ksum, qT, preferred_element_type=jnp.float32)

    C = L // n_chunks
    m_row = jnp.full((1, L), -jnp.inf, dtype=jnp.float32)
    for c in range(n_chunks):
        s_c = jnp.dot(kk[c * C:(c + 1) * C, :], qT,
                      preferred_element_type=jnp.float32)
        m_row = jnp.maximum(m_row, jnp.max(s_c, axis=0, keepdims=True))

    row = (m_row - rowsum * (1.0 / L)) * scale

    lane_iota = jax.lax.broadcasted_iota(jnp.int32, (1, L), 1)
    neg_inf = jnp.float32(-jnp.inf)

    idx_rows = []
    work = row
    for _ in range(k_top):
        i_j = jnp.argmax(work, axis=-1).reshape(1, 1).astype(jnp.int32)
        idx_rows.append(i_j)
        work = jnp.where(lane_iota == i_j, neg_inf, work)
    for _ in range(k_pad - k_top):
        idx_rows.append(jnp.full((1, 1), -1, dtype=jnp.int32))
    idx_col = jnp.concatenate(idx_rows, axis=0)

    onehot = (jax.lax.broadcasted_iota(jnp.int32, (k_pad, L), 1)
              == idx_col).astype(jnp.float32)

    qs = jnp.dot(onehot, q, preferred_element_type=jnp.float32)
    ssp = jax.lax.dot_general(
        qs, kk, (((1,), (1,)), ((), ())),
        preferred_element_type=jnp.float32) * scale
    smax = jnp.max(ssp, axis=-1, keepdims=True)
    e = jnp.exp(ssp - smax)
    att = e / jnp.sum(e, axis=-1, keepdims=True)
    ctx = jnp.dot(att, v, preferred_element_type=jnp.float32)

    v_mean = jnp.mean(v, axis=0, keepdims=True)
    delta = ctx - v_mean
    scat = jax.lax.dot_general(
        onehot, delta, (((0,), (0,)), ((), ())),
        preferred_element_type=jnp.float32)
    out_ref[0] = scat + v_mean


def kernel(Q, K, V):
    B, L, D = Q.shape
    k_top = min(L, max(1, int(_FACTOR * math.log(L + _EPS))))
    k_pad = max(8, ((k_top + 7) // 8) * 8)

    spec = pl.BlockSpec((1, L, D), lambda b: (b, 0, 0))
    return pl.pallas_call(
        functools.partial(_prob_sparse_kernel, k_top=k_top, k_pad=k_pad,
                          L=L, D=D, n_chunks=8),
        grid=(B,),
        in_specs=[spec, spec, spec],
        out_specs=spec,
        out_shape=jax.ShapeDtypeStruct((B, L, D), jnp.float32),
    )(Q, K, V)
